# Initial kernel scaffold; baseline (speedup 1.0000x reference)
import jax
import jax.numpy as jnp
from jax.experimental import pallas as pl


def kernel(x, norm, prev_h, emb_rel, edge_index, edge_type, weight_neighbor, loop_weight):
    raise NotImplementedError("write your pallas kernel here")


# speedup vs baseline: 4.4433x; 4.4433x over previous
import functools

import jax
import jax.numpy as jnp
from jax import lax
from jax.experimental import pallas as pl
from jax.experimental.pallas import tpu as pltpu
from jax.experimental.pallas import tpu_sc as plsc


_LANES = 16
_CHUNK = 128


def _sc_agg_kernel_factory(N, E_pad, D, NC, NS):
    NW = NC * NS
    chunks_per_worker = E_pad // (NW * _CHUNK)
    rows_per_tile = -(-(N + 1) // (NS * _CHUNK)) * _CHUNK
    NP = NS * rows_per_tile
    out_chunks = rows_per_tile // _CHUNK

    mesh = plsc.VectorSubcoreMesh(
        core_axis_name="c", subcore_axis_name="s",
        num_cores=NC, num_subcores=NS)

    @functools.partial(
        pl.kernel,
        mesh=mesh,
        out_type=jax.ShapeDtypeStruct((NC, NP, D), jnp.float32),
        scratch_types=[
            pltpu.MemorySpace.VMEM_SHARED((NP, D), jnp.float32),
            pltpu.MemorySpace.VMEM((_CHUNK,), jnp.int32),
            pltpu.MemorySpace.VMEM((_CHUNK,), jnp.int32),
            pltpu.MemorySpace.VMEM((_CHUNK,), jnp.int32),
            pltpu.MemorySpace.VMEM((_CHUNK, D), jnp.float32),
            pltpu.MemorySpace.VMEM((_CHUNK, D), jnp.float32),
            pltpu.SemaphoreType.DMA,
            pltpu.SemaphoreType.DMA,
        ],
    )
    def sc_agg(x_hbm, rel_hbm, src_hbm, dst_hbm, et_hbm, out_hbm,
               acc_sh, src_v, dst_v, et_v, xrows_v, rrows_v,
               sem1, sem2):
        c = lax.axis_index("c")
        s = lax.axis_index("s")
        wid = s * NC + c

        zeros16 = jnp.zeros((_LANES,), jnp.float32)

        def zrow(j, _):
            for k in range(D // _LANES):
                xrows_v[j, pl.ds(k * _LANES, _LANES)] = zeros16
            return 0

        lax.fori_loop(0, _CHUNK, zrow, 0)
        row0 = s * rows_per_tile
        for cc in range(out_chunks):
            pltpu.sync_copy(xrows_v, acc_sh.at[pl.ds(row0 + cc * _CHUNK, _CHUNK)])
        plsc.subcore_barrier()

        def body(i, _):
            base = (wid * chunks_per_worker + i) * _CHUNK
            pltpu.sync_copy(src_hbm.at[pl.ds(base, _CHUNK)], src_v)
            pltpu.sync_copy(et_hbm.at[pl.ds(base, _CHUNK)], et_v)
            pltpu.sync_copy(dst_hbm.at[pl.ds(base, _CHUNK)], dst_v)
            g1 = pltpu.async_copy(x_hbm.at[src_v], xrows_v, sem1)
            g2 = pltpu.async_copy(rel_hbm.at[et_v], rrows_v, sem2)
            g1.wait()
            g2.wait()

            def mrow(j, _):
                for k in range(D // _LANES):
                    sl = pl.ds(k * _LANES, _LANES)
                    xrows_v[j, sl] = xrows_v[j, sl] * rrows_v[j, sl]
                return 0

            lax.fori_loop(0, _CHUNK, mrow, 0)
            pltpu.sync_copy(xrows_v, acc_sh.at[dst_v], add=True)
            return 0

        lax.fori_loop(0, chunks_per_worker, body, 0)
        plsc.subcore_barrier()

        for cc in range(out_chunks):
            r = row0 + cc * _CHUNK
            pltpu.sync_copy(acc_sh.at[pl.ds(r, _CHUNK)], xrows_v)
            pltpu.sync_copy(xrows_v, out_hbm.at[c, pl.ds(r, _CHUNK)])

    return sc_agg, NP


def _tc_finalize(p0, p1, x, norm, w_n, w_l, block_rows=512):
    N, D = x.shape
    grid = (-(-N // block_rows),)

    def body(p0_ref, p1_ref, x_ref, norm_ref, wn_ref, wl_ref, out_ref):
        s = p0_ref[...] + p1_ref[...]
        agg = jnp.dot(s, wn_ref[...], preferred_element_type=jnp.float32)
        loop = jnp.dot(x_ref[...], wl_ref[...], preferred_element_type=jnp.float32)
        out_ref[...] = agg * norm_ref[...] + loop

    return pl.pallas_call(
        body,
        grid=grid,
        in_specs=[
            pl.BlockSpec((block_rows, D), lambda i: (i, 0)),
            pl.BlockSpec((block_rows, D), lambda i: (i, 0)),
            pl.BlockSpec((block_rows, D), lambda i: (i, 0)),
            pl.BlockSpec((block_rows, 1), lambda i: (i, 0)),
            pl.BlockSpec((D, D), lambda i: (0, 0)),
            pl.BlockSpec((D, D), lambda i: (0, 0)),
        ],
        out_specs=pl.BlockSpec((block_rows, D), lambda i: (i, 0)),
        out_shape=jax.ShapeDtypeStruct((N, D), jnp.float32),
    )(p0, p1, x, norm, w_n, w_l)


def kernel(x, norm, prev_h, emb_rel, edge_index, edge_type, weight_neighbor,
           loop_weight):
    N, D = x.shape
    E = edge_index.shape[1]
    NC, NS = 2, 16
    NW = NC * NS

    E_pad = -(-E // (NW * _CHUNK)) * (NW * _CHUNK)
    pad = E_pad - E
    src = edge_index[0]
    dst = edge_index[1]
    et = edge_type
    if pad:
        src = jnp.concatenate([src, jnp.zeros((pad,), jnp.int32)])
        dst = jnp.concatenate([dst, jnp.full((pad,), N, jnp.int32)])
        et = jnp.concatenate([et, jnp.zeros((pad,), jnp.int32)])

    sc_agg, NP = _sc_agg_kernel_factory(N, E_pad, D, NC, NS)
    partials = sc_agg(x, emb_rel, src, dst, et)
    p0 = partials[0, :N, :]
    p1 = partials[1, :N, :]
    return _tc_finalize(p0, p1, x, norm, weight_neighbor, loop_weight)
